# Initial kernel scaffold; baseline (speedup 1.0000x reference)
import jax
import jax.numpy as jnp
from jax.experimental import pallas as pl


def kernel(Ahat, node_embs, mask, ht, W_map, b_map, Wu, Uu, bu, Wr, Ur, br, Wh, Uh, bh, GCN_init_mapping, W_init, b_init, static_weights):
    raise NotImplementedError("write your pallas kernel here")


# speedup vs baseline: 13.7947x; 13.7947x over previous
import jax
import jax.numpy as jnp
from jax import lax
from jax.experimental import pallas as pl
from jax.experimental.pallas import tpu as pltpu

_B, _N, _D, _R, _K = 4, 4096, 128, 256, 64
_NEG = -3.0e38


def _sem_body(ahat, neT, ht, wmapT, bmap, wu, uu, bu_, wr, ur, br_, wh, uh,
              bh_, gim, winitT, binit, sw,
              ne_out, pol_out, scorer_out, ent_out,
              a1, sem):
    b = pl.program_id(0)
    f32 = jnp.float32
    htr = ht[0]
    net = neT[0]

    scorer = jnp.tanh(
        jnp.dot(htr, wmapT[...], preferred_element_type=f32) + bmap[...])
    snorm = jnp.sqrt(jnp.sum(scorer * scorer))
    scoresT = jnp.dot(scorer, net, preferred_element_type=f32) / snorm

    iota_n = lax.broadcasted_iota(jnp.int32, (1, _N), 1)
    iota_k = lax.broadcasted_iota(jnp.int32, (1, _K), 1)

    ms = scoresT
    vals = jnp.zeros((1, _K), f32)
    idxs = jnp.zeros((1, _K), jnp.int32)
    copies = []
    for i in range(_K):
        m = jnp.max(ms)
        idx = jnp.min(jnp.where(ms == m, iota_n, _N))
        ms = jnp.where(iota_n == idx, _NEG, ms)
        vals = jnp.where(iota_k == i, m, vals)
        idxs = jnp.where(iota_k == i, idx, idxs)
        cp = pltpu.make_async_copy(ahat.at[b, idx], a1.at[i], sem)
        cp.start()
        copies.append(cp)

    smax = jnp.max(scoresT)
    ex = jnp.exp(scoresT - smax)
    z = jnp.sum(ex)
    logz = jnp.log(z)
    ent = logz - jnp.sum(ex * (scoresT - smax)) / z
    pol = jnp.mean(vals) - smax - logz

    iota_row = lax.broadcasted_iota(jnp.int32, (_N, _K), 0)
    ohT = (iota_row == idxs).astype(f32)
    g64T = jnp.dot(net, ohT, preferred_element_type=f32)

    tw = jnp.tanh(vals)
    zc = g64T * tw
    z_topk = jnp.concatenate([zc, zc], axis=1)

    htm = jnp.tanh(
        jnp.dot(htr, winitT[...], preferred_element_type=f32) + binit[...])
    prev_q = lax.dot_general(htm, gim[...], (((0,), (0,)), ((), ())),
                             preferred_element_type=f32)

    upd = jax.nn.sigmoid(
        jnp.dot(wu[...], z_topk, preferred_element_type=f32)
        + jnp.dot(uu[...], prev_q, preferred_element_type=f32) + bu_[...])
    rst = jax.nn.sigmoid(
        jnp.dot(wr[...], z_topk, preferred_element_type=f32)
        + jnp.dot(ur[...], prev_q, preferred_element_type=f32) + br_[...])
    hcap = jnp.tanh(
        jnp.dot(wh[...], z_topk, preferred_element_type=f32)
        + jnp.dot(uh[...], rst * prev_q, preferred_element_type=f32)
        + bh_[...])
    gcn_w = (1.0 - upd) * prev_q + upd * hcap

    for cp in copies:
        cp.wait()
    a1v = a1[...]
    a2 = jnp.dot(a1v, ohT, preferred_element_type=f32)
    colsum = jnp.sum(a2, axis=0, keepdims=True)
    di_row = lax.rsqrt(colsum)
    di_col = jnp.transpose(di_row)
    a2n = a2 * di_row * di_col

    ne_g = jnp.transpose(g64T)
    t1 = jnp.dot(ne_g, gcn_w, preferred_element_type=f32)
    ne2 = jax.nn.relu(jnp.dot(a2n, t1, preferred_element_type=f32))
    t2 = jnp.dot(ne2, sw[...], preferred_element_type=f32)
    ne3 = jax.nn.relu(jnp.dot(a2n, t2, preferred_element_type=f32))

    ne_out[...] = ((ne2 + ne3) * 0.5).reshape(1, _K, _D)
    pol_out[...] = jnp.full((1, 1, 1), pol, f32)
    scorer_out[...] = scorer.reshape(1, 1, _D)
    ent_out[...] = jnp.full((1, 1, 1), ent, f32)


def kernel(Ahat, node_embs, mask, ht, W_map, b_map, Wu, Uu, bu, Wr, Ur, br,
           Wh, Uh, bh, GCN_init_mapping, W_init, b_init, static_weights):
    del mask
    f32 = jnp.float32
    neT = jnp.transpose(node_embs, (0, 2, 1))

    rep = lambda shape: pl.BlockSpec(shape, lambda b: (0,) * len(shape))
    grid_spec = pltpu.PrefetchScalarGridSpec(
        num_scalar_prefetch=0,
        grid=(_B,),
        in_specs=[
            pl.BlockSpec(memory_space=pl.ANY),
            pl.BlockSpec((1, _D, _N), lambda b: (b, 0, 0)),
            pl.BlockSpec((1, 1, _R), lambda b: (b, 0, 0)),
            rep((_R, _D)),
            rep((1, _D)),
            rep((_D, _D)), rep((_D, _D)), rep((_D, _D)),
            rep((_D, _D)), rep((_D, _D)), rep((_D, _D)),
            rep((_D, _D)), rep((_D, _D)), rep((_D, _D)),
            rep((1, _D)),
            rep((_R, _D)),
            rep((1, _D)),
            rep((_D, _D)),
        ],
        out_specs=[
            pl.BlockSpec((1, _K, _D), lambda b: (b, 0, 0)),
            pl.BlockSpec((1, 1, 1), lambda b: (b, 0, 0)),
            pl.BlockSpec((1, 1, _D), lambda b: (b, 0, 0)),
            pl.BlockSpec((1, 1, 1), lambda b: (b, 0, 0)),
        ],
        scratch_shapes=[
            pltpu.VMEM((_K, _N), f32),
            pltpu.SemaphoreType.DMA,
        ],
    )
    ne, pol, scorer, ent = pl.pallas_call(
        _sem_body,
        grid_spec=grid_spec,
        out_shape=[
            jax.ShapeDtypeStruct((_B, _K, _D), f32),
            jax.ShapeDtypeStruct((_B, 1, 1), f32),
            jax.ShapeDtypeStruct((_B, 1, _D), f32),
            jax.ShapeDtypeStruct((_B, 1, 1), f32),
        ],
        compiler_params=pltpu.CompilerParams(
            dimension_semantics=("arbitrary",)),
    )(Ahat, neT, ht.reshape(_B, 1, _R), W_map.T, b_map.reshape(1, _D),
      Wu, Uu, bu, Wr, Ur, br, Wh, Uh, bh, GCN_init_mapping, W_init.T,
      b_init.reshape(1, _D), static_weights)
    return ne, pol.reshape(_B), scorer.reshape(_B, _D), ent.reshape(_B)


# speedup vs baseline: 49.4748x; 3.5865x over previous
import jax
import jax.numpy as jnp
from jax import lax
from jax.experimental import pallas as pl
from jax.experimental.pallas import tpu as pltpu

_B, _N, _D, _R, _K = 4, 4096, 128, 256, 64
_NEG = -3.0e38


def _sem_body(ahat, ne, ht, wmapT, bmap, wu, uu, bu_, wr, ur, br_, wh, uh,
              bh_, gim, winitT, binit, sw,
              ne_out, pol_out, scorer_out, ent_out,
              a1, sem):
    f32 = jnp.float32
    nt = (((1,), (1,)), ((), ()))

    htr = ht[...]
    scorer = jnp.tanh(
        jnp.dot(htr, wmapT[...], preferred_element_type=f32) + bmap[...])
    snorm = jnp.sqrt(jnp.sum(scorer * scorer, axis=1, keepdims=True))

    rows = []
    for b in range(_B):
        rows.append(lax.dot_general(scorer[b:b + 1, :], ne[b], nt,
                                    preferred_element_type=f32))
    scores = jnp.concatenate(rows, axis=0) / snorm

    iota_n = lax.broadcasted_iota(jnp.int32, (_B, _N), 1)
    iota_k = lax.broadcasted_iota(jnp.int32, (1, _K), 1)
    iota_b = lax.broadcasted_iota(jnp.int32, (_B, 1), 0)

    ms = scores
    vals = jnp.zeros((_B, _K), f32)
    idxs = jnp.zeros((_B, _K), jnp.int32)
    copies = []
    for i in range(_K):
        m = jnp.max(ms, axis=1, keepdims=True)
        idxv = jnp.min(jnp.where(ms == m, iota_n, _N),
                       axis=1, keepdims=True)
        ms = jnp.where(iota_n == idxv, _NEG, ms)
        vals = jnp.where(iota_k == i, m, vals)
        idxs = jnp.where(iota_k == i, idxv, idxs)
        for b in range(_B):
            idx_b = jnp.min(jnp.where(iota_b == b, idxv, _N))
            cp = pltpu.make_async_copy(ahat.at[b, idx_b],
                                       a1.at[b * _K + i], sem)
            cp.start()
            copies.append(cp)

    smax = jnp.max(scores, axis=1, keepdims=True)
    ex = jnp.exp(scores - smax)
    z = jnp.sum(ex, axis=1, keepdims=True)
    logz = jnp.log(z)
    ent = logz - jnp.sum(ex * (scores - smax), axis=1, keepdims=True) / z
    pol = jnp.mean(vals, axis=1, keepdims=True) - smax - logz

    htm = jnp.tanh(
        jnp.dot(htr, winitT[...], preferred_element_type=f32) + binit[...])

    iota_row = lax.broadcasted_iota(jnp.int32, (_N, _K), 0)
    iota_lane = lax.broadcasted_iota(jnp.int32, (_K, _N), 1)

    for cp in copies:
        cp.wait()

    for b in range(_B):
        idxs_b = idxs[b:b + 1, :]
        idxs_c = jnp.transpose(idxs_b)
        ohT = (iota_row == idxs_b).astype(f32)
        oh = (iota_lane == idxs_c).astype(f32)

        g64 = jnp.dot(oh, ne[b], preferred_element_type=f32)
        tw = jnp.tanh(jnp.transpose(vals[b:b + 1, :]))
        out64 = g64 * tw
        o64t = jnp.transpose(out64)
        z_topk = jnp.concatenate([o64t, o64t], axis=1)

        prev_q = lax.dot_general(htm[b:b + 1, :], gim[...],
                                 (((0,), (0,)), ((), ())),
                                 preferred_element_type=f32)

        upd = jax.nn.sigmoid(
            jnp.dot(wu[...], z_topk, preferred_element_type=f32)
            + jnp.dot(uu[...], prev_q, preferred_element_type=f32)
            + bu_[...])
        rst = jax.nn.sigmoid(
            jnp.dot(wr[...], z_topk, preferred_element_type=f32)
            + jnp.dot(ur[...], prev_q, preferred_element_type=f32)
            + br_[...])
        hcap = jnp.tanh(
            jnp.dot(wh[...], z_topk, preferred_element_type=f32)
            + jnp.dot(uh[...], rst * prev_q, preferred_element_type=f32)
            + bh_[...])
        gcn_w = (1.0 - upd) * prev_q + upd * hcap

        a1v = a1[b * _K:(b + 1) * _K, :]
        a2 = jnp.dot(a1v, ohT, preferred_element_type=f32)
        colsum = jnp.sum(a2, axis=0, keepdims=True)
        di_row = lax.rsqrt(colsum)
        di_col = jnp.transpose(di_row)
        a2n = a2 * di_row * di_col

        t1 = jnp.dot(g64, gcn_w, preferred_element_type=f32)
        ne2 = jax.nn.relu(jnp.dot(a2n, t1, preferred_element_type=f32))
        t2 = jnp.dot(ne2, sw[...], preferred_element_type=f32)
        ne3 = jax.nn.relu(jnp.dot(a2n, t2, preferred_element_type=f32))

        ne_out[b] = (ne2 + ne3) * 0.5

    pol_out[...] = pol
    scorer_out[...] = scorer
    ent_out[...] = ent


def kernel(Ahat, node_embs, mask, ht, W_map, b_map, Wu, Uu, bu, Wr, Ur, br,
           Wh, Uh, bh, GCN_init_mapping, W_init, b_init, static_weights):
    del mask
    f32 = jnp.float32

    vm = pl.BlockSpec(memory_space=pltpu.VMEM)
    ne, pol, scorer, ent = pl.pallas_call(
        _sem_body,
        in_specs=[pl.BlockSpec(memory_space=pl.ANY)] + [vm] * 17,
        out_specs=[vm, vm, vm, vm],
        out_shape=[
            jax.ShapeDtypeStruct((_B, _K, _D), f32),
            jax.ShapeDtypeStruct((_B, 1), f32),
            jax.ShapeDtypeStruct((_B, _D), f32),
            jax.ShapeDtypeStruct((_B, 1), f32),
        ],
        scratch_shapes=[
            pltpu.VMEM((_B * _K, _N), f32),
            pltpu.SemaphoreType.DMA,
        ],
    )(Ahat, node_embs, ht, W_map.T, b_map.reshape(1, _D), Wu, Uu, bu, Wr,
      Ur, br, Wh, Uh, bh, GCN_init_mapping, W_init.T, b_init.reshape(1, _D),
      static_weights)
    return ne, pol.reshape(_B), scorer, ent.reshape(_B)
